# baseline (speedup 1.0000x reference)
import functools
import jax
import jax.numpy as jnp
from jax import lax
from jax.experimental import pallas as pl
from jax.experimental.pallas import tpu as pltpu
from jax.experimental.pallas import tpu_sc as plsc

_SRC_PAD = 0
_TGT_PAD = 0
_EMBED = 32
_NW = 32
_CHUNK = 128
_L = 16


def _lookup(idx_hbm, table_hbm, out_hbm, idx_v, rows_v, sem, wid, nchunks, pad):
    pltpu.sync_copy(idx_hbm.at[wid], idx_v)
    base_out = wid * (nchunks * _CHUNK)

    def chunk_body(j, carry):
        idx_row = idx_v.at[j]
        pltpu.async_copy(table_hbm.at[idx_row], rows_v, sem).wait()
        zeros = jnp.zeros((_L,), jnp.float32)
        iota = lax.iota(jnp.int32, _L)
        for c in range(_CHUNK // _L):
            idxc = idx_row[pl.ds(c * _L, _L)]
            padmask = idxc == pad
            rowids = c * _L + iota
            for col in range(_EMBED):
                colids = jnp.full((_L,), col, jnp.int32)
                plsc.store_scatter(rows_v, [rowids, colids], zeros,
                                   mask=padmask)

        pltpu.sync_copy(rows_v, out_hbm.at[pl.ds(base_out + j * _CHUNK, _CHUNK)])
        return carry

    lax.fori_loop(0, nchunks, chunk_body, 0)


def _make_kernel(n_flat):
    nchunks = n_flat // (_NW * _CHUNK)

    @functools.partial(
        pl.kernel,
        out_type=(
            jax.ShapeDtypeStruct((n_flat, _EMBED), jnp.float32),
            jax.ShapeDtypeStruct((n_flat, _EMBED), jnp.float32),
        ),
        mesh=plsc.VectorSubcoreMesh(core_axis_name="c", subcore_axis_name="s"),
        compiler_params=pltpu.CompilerParams(
            needs_layout_passes=False, use_tc_tiling_on_sc=False),
        scratch_types=[
            pltpu.VMEM((nchunks, _CHUNK), jnp.int32),
            pltpu.VMEM((_CHUNK, _EMBED), jnp.float32),
            pltpu.SemaphoreType.DMA,
        ],
    )
    def emb_kernel(src_idx, tgt_idx, src_w, tgt_w, src_out, tgt_out,
                   idx_v, rows_v, sem):
        wid = lax.axis_index("s") * 2 + lax.axis_index("c")
        _lookup(src_idx, src_w, src_out, idx_v, rows_v, sem, wid, nchunks,
                _SRC_PAD)
        _lookup(tgt_idx, tgt_w, tgt_out, idx_v, rows_v, sem, wid, nchunks,
                _TGT_PAD)

    return emb_kernel


def kernel(src_indices, tgt_indices, source_weight, target_weight):
    batch, seq = src_indices.shape
    n_flat = batch * seq
    src_r = src_indices.reshape(_NW, -1, _CHUNK).astype(jnp.int32)
    tgt_r = tgt_indices.reshape(_NW, -1, _CHUNK).astype(jnp.int32)
    src_out, tgt_out = _make_kernel(n_flat)(src_r, tgt_r, source_weight,
                                            target_weight)
    return (src_out.reshape(batch, seq, _EMBED),
            tgt_out.reshape(batch, seq, _EMBED))


# speedup vs baseline: 1.0137x; 1.0137x over previous
import functools
import jax
import jax.numpy as jnp
from jax import lax
from jax.experimental import pallas as pl
from jax.experimental.pallas import tpu as pltpu
from jax.experimental.pallas import tpu_sc as plsc

_SRC_PAD = 0
_TGT_PAD = 0
_EMBED = 32
_NW = 32
_CHUNK = 128
_L = 16


def _mask_rows(idx_row, rows, pad):
    zeros = jnp.zeros((_L,), jnp.float32)
    iota = lax.iota(jnp.int32, _L)
    for c in range(_CHUNK // _L):
        idxc = idx_row[pl.ds(c * _L, _L)]
        padmask = idxc == pad
        rowids = c * _L + iota
        for col in range(_EMBED):
            colids = jnp.full((_L,), col, jnp.int32)
            plsc.store_scatter(rows, [rowids, colids], zeros, mask=padmask)


def _lookup(idx_hbm, table_hbm, out_hbm, idx_v, rows0, rows1,
            gsem0, gsem1, wsem0, wsem1, wid, nchunks, pad):
    pltpu.sync_copy(idx_hbm.at[wid], idx_v)
    base_out = wid * (nchunks * _CHUNK)
    npairs = nchunks // 2

    def gather(j, rows, gsem):
        pltpu.async_copy(table_hbm.at[idx_v.at[j]], rows, gsem)

    def wait_gather(j, rows, gsem):
        pltpu.make_async_copy(table_hbm.at[idx_v.at[j]], rows, gsem).wait()

    def write(j, rows, wsem):
        pltpu.async_copy(
            rows, out_hbm.at[pl.ds(base_out + j * _CHUNK, _CHUNK)], wsem)

    def wait_write(j, rows, wsem):
        pltpu.make_async_copy(
            rows, out_hbm.at[pl.ds(base_out + j * _CHUNK, _CHUNK)],
            wsem).wait()

    gather(0, rows0, gsem0)

    def pair(jj, carry):
        j0 = 2 * jj
        j1 = 2 * jj + 1
        wait_gather(j0, rows0, gsem0)
        _mask_rows(idx_v.at[j0], rows0, pad)
        write(j0, rows0, wsem0)

        @pl.when(jj >= 1)
        def _():
            wait_write(j0 - 1, rows1, wsem1)

        gather(j1, rows1, gsem1)

        wait_gather(j1, rows1, gsem1)
        _mask_rows(idx_v.at[j1], rows1, pad)
        write(j1, rows1, wsem1)
        wait_write(j0, rows0, wsem0)

        @pl.when(jj <= npairs - 2)
        def _():
            gather(j1 + 1, rows0, gsem0)

        return carry

    lax.fori_loop(0, npairs, pair, 0)
    wait_write(nchunks - 1, rows1, wsem1)


def _make_kernel(n_flat):
    nchunks = n_flat // (_NW * _CHUNK)

    @functools.partial(
        pl.kernel,
        out_type=(
            jax.ShapeDtypeStruct((n_flat, _EMBED), jnp.float32),
            jax.ShapeDtypeStruct((n_flat, _EMBED), jnp.float32),
        ),
        mesh=plsc.VectorSubcoreMesh(core_axis_name="c", subcore_axis_name="s"),
        compiler_params=pltpu.CompilerParams(
            needs_layout_passes=False, use_tc_tiling_on_sc=False),
        scratch_types=[
            pltpu.VMEM((nchunks, _CHUNK), jnp.int32),
            pltpu.VMEM((_CHUNK, _EMBED), jnp.float32),
            pltpu.VMEM((_CHUNK, _EMBED), jnp.float32),
            pltpu.SemaphoreType.DMA,
            pltpu.SemaphoreType.DMA,
            pltpu.SemaphoreType.DMA,
            pltpu.SemaphoreType.DMA,
        ],
    )
    def emb_kernel(src_idx, tgt_idx, src_w, tgt_w, src_out, tgt_out,
                   idx_v, rows0, rows1, gsem0, gsem1, wsem0, wsem1):
        wid = lax.axis_index("s") * 2 + lax.axis_index("c")
        _lookup(src_idx, src_w, src_out, idx_v, rows0, rows1,
                gsem0, gsem1, wsem0, wsem1, wid, nchunks, _SRC_PAD)
        _lookup(tgt_idx, tgt_w, tgt_out, idx_v, rows0, rows1,
                gsem0, gsem1, wsem0, wsem1, wid, nchunks, _TGT_PAD)

    return emb_kernel


def kernel(src_indices, tgt_indices, source_weight, target_weight):
    batch, seq = src_indices.shape
    n_flat = batch * seq
    src_r = src_indices.reshape(_NW, -1, _CHUNK).astype(jnp.int32)
    tgt_r = tgt_indices.reshape(_NW, -1, _CHUNK).astype(jnp.int32)
    src_out, tgt_out = _make_kernel(n_flat)(src_r, tgt_r, source_weight,
                                            target_weight)
    return (src_out.reshape(batch, seq, _EMBED),
            tgt_out.reshape(batch, seq, _EMBED))


# speedup vs baseline: 1.1926x; 1.1765x over previous
import functools
import jax
import jax.numpy as jnp
from jax import lax
from jax.experimental import pallas as pl
from jax.experimental.pallas import tpu as pltpu
from jax.experimental.pallas import tpu_sc as plsc

_SRC_PAD = 0
_TGT_PAD = 0
_EMBED = 32
_NW = 32
_BT = 128
_L = 16


def _transpose_chunk(idx_row, rows_v, t_v, pad):
    for g in range(_BT // _L):
        idxc = idx_row[pl.ds(g * _L, _L)]
        m = jnp.where(idxc == pad, 0.0, 1.0).astype(jnp.float32)
        rid = g * _L + lax.iota(jnp.int32, _L)
        for e in range(_EMBED):
            cid = jnp.full((_L,), e, jnp.int32)
            vals = plsc.load_gather(rows_v, [rid, cid])
            t_v[e // 8, e % 8, pl.ds(g * _L, _L)] = vals * m


def _lookup(idx_hbm, table_hbm, out_hbm, idx_v, rows0, rows1, t0, t1,
            gsem0, gsem1, wsem0, wsem1, wid, nseq, pad):
    pltpu.sync_copy(idx_hbm.at[:, pl.ds(wid * _BT, _BT)], idx_v)
    npairs = nseq // 2

    def gather(s, rows, gsem):
        pltpu.async_copy(table_hbm.at[idx_v.at[s]], rows, gsem)

    def wait_gather(s, rows, gsem):
        pltpu.make_async_copy(table_hbm.at[idx_v.at[s]], rows, gsem).wait()

    def write(s, t_v, wsem):
        for eh in range(4):
            pltpu.async_copy(t_v.at[eh], out_hbm.at[s, eh, wid], wsem)

    def wait_write(s, t_v, wsem):
        for eh in range(4):
            pltpu.make_async_copy(
                t_v.at[eh], out_hbm.at[s, eh, wid], wsem).wait()

    gather(0, rows0, gsem0)

    def pair(jj, carry):
        s0 = 2 * jj
        s1 = 2 * jj + 1
        wait_gather(s0, rows0, gsem0)
        gather(s1, rows1, gsem1)

        @pl.when(jj >= 1)
        def _():
            wait_write(s0 - 2, t0, wsem0)

        _transpose_chunk(idx_v.at[s0], rows0, t0, pad)
        write(s0, t0, wsem0)

        wait_gather(s1, rows1, gsem1)

        @pl.when(jj <= npairs - 2)
        def _():
            gather(s1 + 1, rows0, gsem0)

        @pl.when(jj >= 1)
        def _():
            wait_write(s1 - 2, t1, wsem1)

        _transpose_chunk(idx_v.at[s1], rows1, t1, pad)
        write(s1, t1, wsem1)
        return carry

    lax.fori_loop(0, npairs, pair, 0)
    wait_write(nseq - 2, t0, wsem0)
    wait_write(nseq - 1, t1, wsem1)


def _make_kernel(nseq, nbatch):
    @functools.partial(
        pl.kernel,
        out_type=(
            jax.ShapeDtypeStruct((nseq, 4, _NW, 8, _BT), jnp.float32),
            jax.ShapeDtypeStruct((nseq, 4, _NW, 8, _BT), jnp.float32),
        ),
        mesh=plsc.VectorSubcoreMesh(core_axis_name="c", subcore_axis_name="s"),
        compiler_params=pltpu.CompilerParams(
            needs_layout_passes=False, use_tc_tiling_on_sc=False),
        scratch_types=[
            pltpu.VMEM((nseq, _BT), jnp.int32),
            pltpu.VMEM((_BT, _EMBED), jnp.float32),
            pltpu.VMEM((_BT, _EMBED), jnp.float32),
            pltpu.VMEM((4, 8, _BT), jnp.float32),
            pltpu.VMEM((4, 8, _BT), jnp.float32),
            pltpu.SemaphoreType.DMA,
            pltpu.SemaphoreType.DMA,
            pltpu.SemaphoreType.DMA,
            pltpu.SemaphoreType.DMA,
        ],
    )
    def emb_kernel(src_idx, tgt_idx, src_w, tgt_w, src_out, tgt_out,
                   idx_v, rows0, rows1, t0, t1, gsem0, gsem1, wsem0, wsem1):
        wid = lax.axis_index("s") * 2 + lax.axis_index("c")
        _lookup(src_idx, src_w, src_out, idx_v, rows0, rows1, t0, t1,
                gsem0, gsem1, wsem0, wsem1, wid, nseq, _SRC_PAD)
        _lookup(tgt_idx, tgt_w, tgt_out, idx_v, rows0, rows1, t0, t1,
                gsem0, gsem1, wsem0, wsem1, wid, nseq, _TGT_PAD)

    return emb_kernel


def kernel(src_indices, tgt_indices, source_weight, target_weight):
    batch, seq = src_indices.shape
    src_t = jnp.transpose(src_indices).astype(jnp.int32)
    tgt_t = jnp.transpose(tgt_indices).astype(jnp.int32)
    src5, tgt5 = _make_kernel(seq, batch)(src_t, tgt_t, source_weight,
                                          target_weight)
    src_out = src5.transpose(2, 4, 0, 1, 3).reshape(batch, seq, _EMBED)
    tgt_out = tgt5.transpose(2, 4, 0, 1, 3).reshape(batch, seq, _EMBED)
    return (src_out, tgt_out)


# speedup vs baseline: 1.1979x; 1.0044x over previous
import functools
import jax
import jax.numpy as jnp
from jax import lax
from jax.experimental import pallas as pl
from jax.experimental.pallas import tpu as pltpu
from jax.experimental.pallas import tpu_sc as plsc

_SRC_PAD = 0
_TGT_PAD = 0
_EMBED = 32
_NW = 32
_BT = 128
_L = 16


def _mask_rows(idx_row, rows_v, pad):
    zeros = jnp.zeros((_L,), jnp.float32)
    iota = lax.iota(jnp.int32, _L)
    for g in range(_BT // _L):
        idxc = idx_row[pl.ds(g * _L, _L)]
        padmask = idxc == pad
        rowids = g * _L + iota
        for col in range(_EMBED):
            colids = jnp.full((_L,), col, jnp.int32)
            plsc.store_scatter(rows_v, [rowids, colids], zeros, mask=padmask)


def _lookup(idx_hbm, table_hbm, out_hbm, idx_raw, idx_v, rows0, rows1,
            gsem0, gsem1, wsem0, wsem1, wid, nseq, pad):
    pltpu.sync_copy(idx_hbm.at[pl.ds(wid * _BT, _BT)], idx_raw)

    def idx_t_body(s, carry):
        cid = jnp.full((_L,), s, jnp.int32)
        for g in range(_BT // _L):
            rid = g * _L + lax.iota(jnp.int32, _L)
            idx_v[s, pl.ds(g * _L, _L)] = plsc.load_gather(idx_raw, [rid, cid])
        return carry

    lax.fori_loop(0, nseq, idx_t_body, 0)
    npairs = nseq // 2

    def gather(s, rows, gsem):
        pltpu.async_copy(table_hbm.at[idx_v.at[s]], rows, gsem)

    def wait_gather(s, rows, gsem):
        pltpu.make_async_copy(table_hbm.at[idx_v.at[s]], rows, gsem).wait()

    def write(s, rows, wsem):
        pltpu.async_copy(rows, out_hbm.at[pl.ds(wid * _BT, _BT), s], wsem)

    def wait_write(s, rows, wsem):
        pltpu.make_async_copy(
            rows, out_hbm.at[pl.ds(wid * _BT, _BT), s], wsem).wait()

    gather(0, rows0, gsem0)

    def pair(jj, carry):
        s0 = 2 * jj
        s1 = 2 * jj + 1
        wait_gather(s0, rows0, gsem0)
        _mask_rows(idx_v.at[s0], rows0, pad)

        @pl.when(jj >= 1)
        def _():
            wait_write(s1 - 2, rows1, wsem1)

        gather(s1, rows1, gsem1)
        write(s0, rows0, wsem0)

        wait_gather(s1, rows1, gsem1)
        _mask_rows(idx_v.at[s1], rows1, pad)

        @pl.when(jj <= npairs - 2)
        def _():
            wait_write(s0, rows0, wsem0)
            gather(s0 + 2, rows0, gsem0)

        write(s1, rows1, wsem1)
        return carry

    lax.fori_loop(0, npairs, pair, 0)
    wait_write(nseq - 2, rows0, wsem0)
    wait_write(nseq - 1, rows1, wsem1)


def _make_kernel(nseq, nbatch):
    @functools.partial(
        pl.kernel,
        out_type=(
            jax.ShapeDtypeStruct((nbatch, nseq, _EMBED), jnp.float32),
            jax.ShapeDtypeStruct((nbatch, nseq, _EMBED), jnp.float32),
        ),
        mesh=plsc.VectorSubcoreMesh(core_axis_name="c", subcore_axis_name="s"),
        compiler_params=pltpu.CompilerParams(
            needs_layout_passes=False, use_tc_tiling_on_sc=False),
        scratch_types=[
            pltpu.VMEM((_BT, nseq), jnp.int32),
            pltpu.VMEM((nseq, _BT), jnp.int32),
            pltpu.VMEM((_BT, _EMBED), jnp.float32),
            pltpu.VMEM((_BT, _EMBED), jnp.float32),
            pltpu.SemaphoreType.DMA,
            pltpu.SemaphoreType.DMA,
            pltpu.SemaphoreType.DMA,
            pltpu.SemaphoreType.DMA,
        ],
    )
    def emb_kernel(src_idx, tgt_idx, src_w, tgt_w, src_out, tgt_out,
                   idx_raw, idx_v, rows0, rows1, gsem0, gsem1, wsem0, wsem1):
        wid = lax.axis_index("s") * 2 + lax.axis_index("c")
        _lookup(src_idx, src_w, src_out, idx_raw, idx_v, rows0, rows1,
                gsem0, gsem1, wsem0, wsem1, wid, nseq, _SRC_PAD)
        _lookup(tgt_idx, tgt_w, tgt_out, idx_raw, idx_v, rows0, rows1,
                gsem0, gsem1, wsem0, wsem1, wid, nseq, _TGT_PAD)

    return emb_kernel


def kernel(src_indices, tgt_indices, source_weight, target_weight):
    batch, seq = src_indices.shape
    return _make_kernel(seq, batch)(src_indices, tgt_indices,
                                    source_weight, target_weight)


# speedup vs baseline: 1.3525x; 1.1291x over previous
import functools
import jax
import jax.numpy as jnp
from jax import lax
from jax.experimental import pallas as pl
from jax.experimental.pallas import tpu as pltpu
from jax.experimental.pallas import tpu_sc as plsc

_SRC_PAD = 0
_TGT_PAD = 0
_EMBED = 32
_NW = 32
_BT = 128
_L = 16


def _mask_rows(idx_row, rows_v, pad):
    zeros = jnp.zeros((_L,), jnp.float32)
    iota = lax.iota(jnp.int32, _L)
    for g in range(_BT // _L):
        idxc = idx_row[pl.ds(g * _L, _L)]
        padmask = idxc == pad
        rowids = g * _L + iota
        for col in range(_EMBED):
            colids = jnp.full((_L,), col, jnp.int32)
            plsc.store_scatter(rows_v, [rowids, colids], zeros, mask=padmask)


def _lookup(idx_hbm, table_hbm, out_hbm, idx_raw, idx_v, rows0, rows1,
            gsem0, gsem1, wsem0, wsem1, wid, nseq, pad):
    pltpu.sync_copy(idx_hbm.at[pl.ds(wid * _BT, _BT)], idx_raw)

    def idx_t_body(s, carry):
        cid = jnp.full((_L,), s, jnp.int32)
        for g in range(_BT // _L):
            rid = g * _L + lax.iota(jnp.int32, _L)
            idx_v[s, pl.ds(g * _L, _L)] = plsc.load_gather(idx_raw, [rid, cid])
        return carry

    lax.fori_loop(0, nseq, idx_t_body, 0)
    npairs = nseq // 2

    def gather(s, rows, gsem):
        pltpu.async_copy(table_hbm.at[idx_v.at[s]], rows, gsem)

    def wait_gather(s, rows, gsem):
        pltpu.make_async_copy(table_hbm.at[idx_v.at[s]], rows, gsem).wait()

    def write(s, rows, wsem):
        pltpu.async_copy(rows, out_hbm.at[pl.ds(wid * _BT, _BT), s], wsem)

    def wait_write(s, rows, wsem):
        pltpu.make_async_copy(
            rows, out_hbm.at[pl.ds(wid * _BT, _BT), s], wsem).wait()

    gather(0, rows0, gsem0)

    def pair(jj, carry):
        s0 = 2 * jj
        s1 = 2 * jj + 1
        wait_gather(s0, rows0, gsem0)
        _mask_rows(idx_v.at[s0], rows0, pad)

        @pl.when(jj >= 1)
        def _():
            wait_write(s1 - 2, rows1, wsem1)

        gather(s1, rows1, gsem1)
        write(s0, rows0, wsem0)

        wait_gather(s1, rows1, gsem1)
        _mask_rows(idx_v.at[s1], rows1, pad)

        @pl.when(jj <= npairs - 2)
        def _():
            wait_write(s0, rows0, wsem0)
            gather(s0 + 2, rows0, gsem0)

        write(s1, rows1, wsem1)
        return carry

    lax.fori_loop(0, npairs, pair, 0)
    wait_write(nseq - 2, rows0, wsem0)
    wait_write(nseq - 1, rows1, wsem1)


def _make_kernel(nseq, nbatch, pad):
    @functools.partial(
        pl.kernel,
        out_type=jax.ShapeDtypeStruct((nbatch, nseq, _EMBED), jnp.float32),
        mesh=plsc.VectorSubcoreMesh(core_axis_name="c", subcore_axis_name="s"),
        compiler_params=pltpu.CompilerParams(
            needs_layout_passes=False, use_tc_tiling_on_sc=False),
        scratch_types=[
            pltpu.VMEM((_BT, nseq), jnp.int32),
            pltpu.VMEM((nseq, _BT), jnp.int32),
            pltpu.VMEM((_BT, _EMBED), jnp.float32),
            pltpu.VMEM((_BT, _EMBED), jnp.float32),
            pltpu.SemaphoreType.DMA,
            pltpu.SemaphoreType.DMA,
            pltpu.SemaphoreType.DMA,
            pltpu.SemaphoreType.DMA,
        ],
    )
    def emb_kernel(idx, table, out, idx_raw, idx_v, rows0, rows1,
                   gsem0, gsem1, wsem0, wsem1):
        wid = lax.axis_index("s") * 2 + lax.axis_index("c")
        _lookup(idx, table, out, idx_raw, idx_v, rows0, rows1,
                gsem0, gsem1, wsem0, wsem1, wid, nseq, pad)

    return emb_kernel


def kernel(src_indices, tgt_indices, source_weight, target_weight):
    batch, seq = src_indices.shape
    src_out = _make_kernel(seq, batch, _SRC_PAD)(src_indices, source_weight)
    tgt_out = _make_kernel(seq, batch, _TGT_PAD)(tgt_indices, target_weight)
    return (src_out, tgt_out)
